# baseline (speedup 1.0000x reference)
import functools

import jax
import jax.numpy as jnp
from jax import lax
from jax.experimental import pallas as pl

B = 4096
V = 1000
K = 128
OUT_DIM = 2 * V - 1

VP = 1024
ROWS = 8

def _i32(x):
    return x - (1 << 32) if x >= (1 << 31) else x

S1 = (_i32(1832780943), _i32(270669613))
S2 = (_i32(64467757), _i32(2916123636))

_TINY = float.fromhex('0x1.0p-126')

_ROT = ((13, 15, 26, 6), (17, 29, 16, 24))


def _rotl(x, r):
    return lax.shift_left(x, jnp.int32(r)) | lax.shift_right_logical(
        x, jnp.int32(32 - r))


def _threefry_bits(k1, k2, x1):
    mask = 0xFFFFFFFF
    ks0 = k1 & mask
    ks1 = k2 & mask
    ks2 = ks0 ^ ks1 ^ 0x1BD11BDA
    ks = (ks0, ks1, ks2)
    x0 = jnp.int32(_i32(ks0))
    for i in range(5):
        for r in _ROT[i % 2]:
            x0 = x0 + x1
            x1 = _rotl(x1, r) ^ x0
        x0 = x0 + jnp.int32(_i32(ks[(i + 1) % 3]))
        x1 = x1 + jnp.int32(_i32((ks[(i + 2) % 3] + (i + 1)) & mask))
    return x0 ^ x1


def _sample_one(p_row, key, row_c, n0, vvf):
    psum = jnp.sum(p_row, axis=1, keepdims=True)
    inv_pn = jnp.float32(0.6931471805599453) / (
        p_row / psum + jnp.float32(1e-12))
    x1 = n0 + (row_c + jnp.int32(key[1]))
    bits = _threefry_bits(key[0], key[1], x1)
    f = lax.bitcast_convert_type(
        lax.shift_right_logical(bits, jnp.int32(9)) | jnp.int32(0x3F800000),
        jnp.float32)
    u = (f - jnp.float32(1.0)) + jnp.float32(_TINY)
    w = jnp.log2(u) * inv_pn
    m = jnp.max(w, axis=1, keepdims=True)
    oh = jnp.where(w == m, jnp.float32(1.0), jnp.float32(0.0))
    idx = lax.dot_general(oh, vvf, (((1,), (1,)), ((), ())),
                          preferred_element_type=jnp.float32)
    g = lax.dot_general(oh, p_row, (((1,), (1,)), ((), ())),
                        preferred_element_type=jnp.float32)
    return idx, g


def _ised_kernel(p1_ref, p2_ref, n0_ref, vvf_ref, lane_ref, out_ref):
    base = pl.program_id(0) * ROWS
    n0 = n0_ref[...]
    vvf = vvf_ref[...]
    lane = lane_ref[...]
    for r in range(ROWS):
        row_c = (base + r) * V
        idx1, g1 = _sample_one(p1_ref[r], S1, row_c, n0, vvf)
        idx2, g2 = _sample_one(p2_ref[r], S2, row_c, n0, vvf)

        ridx = (idx1 + idx2).astype(jnp.int32)
        probs = g1 * g2
        hi = lax.shift_right_logical(ridx, jnp.int32(7))
        lo = ridx & jnp.int32(127)
        a = jnp.where(hi == lane, probs, jnp.float32(0.0))
        bm = jnp.where(lo == lane, jnp.float32(1.0), jnp.float32(0.0))
        y = lax.dot_general(a, bm, (((0,), (0,)), ((), ())),
                            preferred_element_type=jnp.float32)
        y16 = y[:16]
        norm = jnp.sum(y16)
        out_ref[r] = y16 / jnp.maximum(norm, jnp.float32(1e-12))


def _run_block(p1p, p2p, n0, vvf, lane):
    nb = p1p.shape[0]
    return pl.pallas_call(
        _ised_kernel,
        grid=(nb // ROWS,),
        in_specs=[
            pl.BlockSpec((ROWS, 1, VP), lambda i: (i, 0, 0)),
            pl.BlockSpec((ROWS, 1, VP), lambda i: (i, 0, 0)),
            pl.BlockSpec((K, VP), lambda i: (0, 0)),
            pl.BlockSpec((1, VP), lambda i: (0, 0)),
            pl.BlockSpec((1, 128), lambda i: (0, 0)),
        ],
        out_specs=pl.BlockSpec((ROWS, 16, 128), lambda i: (i, 0, 0)),
        out_shape=jax.ShapeDtypeStruct((nb, 16, 128), jnp.float32),
    )(p1p, p2p, n0, vvf, lane)


@functools.partial(jax.jit, static_argnames=())
def kernel(p1, p2):
    p1p = jnp.pad(p1, ((0, 0), (0, VP - V))).reshape(B, 1, VP)
    p2p = jnp.pad(p2, ((0, 0), (0, VP - V))).reshape(B, 1, VP)
    n0 = (jnp.arange(K, dtype=jnp.int32) * jnp.int32(B * V))[:, None] \
        + jnp.arange(VP, dtype=jnp.int32)[None, :]
    vvf = jnp.arange(VP, dtype=jnp.float32)[None, :]
    lane = jnp.arange(128, dtype=jnp.int32)[None, :]

    devs = jax.devices()
    nd = 2 if len(devs) >= 2 and B % (2 * ROWS) == 0 else 1
    if nd > 1:
        import numpy as np
        from jax.sharding import Mesh, PartitionSpec as P

        def body(p1s, p2s, n0s, vvs, lanes):
            off = lax.axis_index("d").astype(jnp.int32) \
                * jnp.int32((B // nd) * V)
            return _run_block(p1s, p2s, n0s + off, vvs, lanes)

        yp = jax.shard_map(
            body,
            mesh=Mesh(np.array(devs[:nd]), ("d",)),
            in_specs=(P("d"), P("d"), P(), P(), P()),
            out_specs=P("d"),
            check_vma=False,
        )(p1p, p2p, n0, vvf, lane)
    else:
        yp = _run_block(p1p, p2p, n0, vvf, lane)
    return yp.reshape(B, 2048)[:, :OUT_DIM]


# speedup vs baseline: 1.3112x; 1.3112x over previous
import functools

import jax
import jax.numpy as jnp
from jax import lax
from jax.experimental import pallas as pl

B = 4096
V = 1000
K = 128
OUT_DIM = 2 * V - 1

VP = 1024
ROWS = 4

def _i32(x):
    return x - (1 << 32) if x >= (1 << 31) else x

S1 = (_i32(1832780943), _i32(270669613))
S2 = (_i32(64467757), _i32(2916123636))

_TINY = float.fromhex('0x1.0p-126')

_ROT = ((13, 15, 26, 6), (17, 29, 16, 24))


def _rotl(x, r):
    return lax.shift_left(x, jnp.int32(r)) | lax.shift_right_logical(
        x, jnp.int32(32 - r))


def _threefry_bits(k1, k2, x1):
    mask = 0xFFFFFFFF
    ks0 = k1 & mask
    ks1 = k2 & mask
    ks2 = ks0 ^ ks1 ^ 0x1BD11BDA
    ks = (ks0, ks1, ks2)
    x0 = jnp.int32(_i32(ks0))
    for i in range(5):
        for r in _ROT[i % 2]:
            x0 = x0 + x1
            x1 = _rotl(x1, r) ^ x0
        x0 = x0 + jnp.int32(_i32(ks[(i + 1) % 3]))
        x1 = x1 + jnp.int32(_i32((ks[(i + 2) % 3] + (i + 1)) & mask))
    return x0 ^ x1


def _sample_one(p_row, key, row_c, n0, vvf):
    psum = jnp.sum(p_row, axis=1, keepdims=True)
    inv_pn = jnp.float32(0.6931471805599453) / (
        p_row / psum + jnp.float32(1e-12))
    x1 = n0 + (row_c + jnp.int32(key[1]))
    bits = _threefry_bits(key[0], key[1], x1)
    f = lax.bitcast_convert_type(
        lax.shift_right_logical(bits, jnp.int32(9)) | jnp.int32(0x3F800000),
        jnp.float32)
    u = (f - jnp.float32(1.0)) + jnp.float32(_TINY)
    w = jnp.log2(u) * inv_pn
    m = jnp.max(w, axis=1, keepdims=True)
    oh = jnp.where(w == m, jnp.float32(1.0), jnp.float32(0.0))
    idx = lax.dot_general(oh, vvf, (((1,), (1,)), ((), ())),
                          preferred_element_type=jnp.float32)
    g = lax.dot_general(oh, p_row, (((1,), (1,)), ((), ())),
                        preferred_element_type=jnp.float32)
    return idx, g


def _ised_kernel(p1_ref, p2_ref, n0_ref, vvf_ref, lane_ref, out_ref):
    base = pl.program_id(0) * ROWS
    n0 = n0_ref[...]
    vvf = vvf_ref[...]
    lane = lane_ref[...]
    for r in range(ROWS):
        row_c = (base + r) * V
        idx1, g1 = _sample_one(p1_ref[r], S1, row_c, n0, vvf)
        idx2, g2 = _sample_one(p2_ref[r], S2, row_c, n0, vvf)

        ridx = (idx1 + idx2).astype(jnp.int32)
        probs = g1 * g2
        hi = lax.shift_right_logical(ridx, jnp.int32(7))
        lo = ridx & jnp.int32(127)
        a = jnp.where(hi == lane, probs, jnp.float32(0.0))
        bm = jnp.where(lo == lane, jnp.float32(1.0), jnp.float32(0.0))
        y = lax.dot_general(a, bm, (((0,), (0,)), ((), ())),
                            preferred_element_type=jnp.float32)
        y16 = y[:16]
        norm = jnp.sum(y16)
        out_ref[r] = y16 / jnp.maximum(norm, jnp.float32(1e-12))


def _run_block(p1p, p2p, n0, vvf, lane):
    nb = p1p.shape[0]
    return pl.pallas_call(
        _ised_kernel,
        grid=(nb // ROWS,),
        in_specs=[
            pl.BlockSpec((ROWS, 1, VP), lambda i: (i, 0, 0)),
            pl.BlockSpec((ROWS, 1, VP), lambda i: (i, 0, 0)),
            pl.BlockSpec((K, VP), lambda i: (0, 0)),
            pl.BlockSpec((1, VP), lambda i: (0, 0)),
            pl.BlockSpec((1, 128), lambda i: (0, 0)),
        ],
        out_specs=pl.BlockSpec((ROWS, 16, 128), lambda i: (i, 0, 0)),
        out_shape=jax.ShapeDtypeStruct((nb, 16, 128), jnp.float32),
    )(p1p, p2p, n0, vvf, lane)


@functools.partial(jax.jit, static_argnames=())
def kernel(p1, p2):
    p1p = jnp.pad(p1, ((0, 0), (0, VP - V))).reshape(B, 1, VP)
    p2p = jnp.pad(p2, ((0, 0), (0, VP - V))).reshape(B, 1, VP)
    n0 = (jnp.arange(K, dtype=jnp.int32) * jnp.int32(B * V))[:, None] \
        + jnp.arange(VP, dtype=jnp.int32)[None, :]
    vvf = jnp.arange(VP, dtype=jnp.float32)[None, :]
    lane = jnp.arange(128, dtype=jnp.int32)[None, :]

    devs = jax.devices()
    nd = 2 if len(devs) >= 2 and B % (2 * ROWS) == 0 else 1
    if nd > 1:
        import numpy as np
        from jax.sharding import Mesh, PartitionSpec as P

        def body(p1s, p2s, n0s, vvs, lanes):
            off = lax.axis_index("d").astype(jnp.int32) \
                * jnp.int32((B // nd) * V)
            return _run_block(p1s, p2s, n0s + off, vvs, lanes)

        yp = jax.shard_map(
            body,
            mesh=Mesh(np.array(devs[:nd]), ("d",)),
            in_specs=(P("d"), P("d"), P(), P(), P()),
            out_specs=P("d"),
            check_vma=False,
        )(p1p, p2p, n0, vvf, lane)
    else:
        yp = _run_block(p1p, p2p, n0, vvf, lane)
    return yp.reshape(B, 2048)[:, :OUT_DIM]
